# baseline (speedup 1.0000x reference)
import functools

import jax
import jax.numpy as jnp
from jax import lax
from jax.experimental import pallas as pl
from jax.experimental.pallas import tpu as pltpu
from jax.experimental.pallas import tpu_sc as plsc

B, C, N, R = 8, 64, 32768, 32
NV = R * R * R
LANES = 16
NVEC = N // LANES
CHUNK = 8192
NCHUNK = N // CHUNK
SPLITS = 2
HB = B // SPLITS
NTEC = 32
TPB = NTEC // HB
CPG = C // TPB
U = 8


def _norm_body(coords_ref, nc_ref, h_ref):
    c = coords_ref[...]
    mean = jnp.mean(c, axis=2, keepdims=True)
    x = c - mean
    sumsq = jnp.sum(x * x, axis=0)
    denom = jnp.sqrt(jnp.max(sumsq, axis=1, keepdims=True)) * 2.0
    nc = x / denom[None] + 0.5
    nc = jnp.clip(nc * float(R), 0.0, float(R - 1))
    nc_ref[...] = nc
    vox = jnp.round(nc).astype(jnp.int32)
    h_ref[...] = vox[0] * (R * R) + vox[1] * R + vox[2]


def _stage_norm(coords_t):
    return pl.pallas_call(
        _norm_body,
        grid=(1,),
        in_specs=[pl.BlockSpec((3, B, N), lambda i: (0, 0, 0))],
        out_specs=[
            pl.BlockSpec((3, B, N), lambda i: (0, 0, 0)),
            pl.BlockSpec((B, N), lambda i: (0, 0)),
        ],
        out_shape=[
            jax.ShapeDtypeStruct((3, B, N), jnp.float32),
            jax.ShapeDtypeStruct((B, N), jnp.int32),
        ],
    )(coords_t)


_mesh = plsc.VectorSubcoreMesh(core_axis_name="c", subcore_axis_name="s")


def _make_sc_scatter(half):
    @functools.partial(
        pl.kernel,
        mesh=_mesh,
        out_type=[
            jax.ShapeDtypeStruct((HB * C, N), jnp.float32),
            jax.ShapeDtypeStruct((HB, NV), jnp.float32),
        ],
        compiler_params=pltpu.CompilerParams(needs_layout_passes=False),
        scratch_types=[
            pltpu.VMEM((N,), jnp.int32),
            pltpu.VMEM((CHUNK,), jnp.float32),
            pltpu.VMEM((CHUNK,), jnp.float32),
            pltpu.VMEM((NV,), jnp.float32),
            pltpu.VMEM((NV,), jnp.float32),
            pltpu.SemaphoreType.DMA,
            pltpu.SemaphoreType.DMA,
            pltpu.SemaphoreType.DMA,
            pltpu.SemaphoreType.DMA,
        ],
    )
    def sc_scatter(feat_hbm, h_hbm, sums_hbm, counts_hbm, idx_v, feat0_v,
                   feat1_v, acc0_v, acc1_v, fsem0, fsem1, osem0, osem1):
        wid = lax.axis_index("s") * 2 + lax.axis_index("c")
        b = wid // TPB
        cg = wid % TPB
        feats = [feat0_v, feat1_v]
        accs = [acc0_v, acc1_v]
        fsem = [fsem0, fsem1]
        osem = [osem0, osem1]

        pltpu.sync_copy(h_hbm.at[half * HB + b], idx_v)

        zeros16 = jnp.zeros((LANES,), jnp.float32)
        ones16 = jnp.ones((LANES,), jnp.float32)

        def _zero(ref):
            def body(i, _):
                for u in range(U):
                    ref[pl.ds((i * U + u) * LANES, LANES)] = zeros16
                return 0

            lax.fori_loop(0, NVEC // U, body, 0)

        _zero(acc0_v)

        @pl.when(cg == b % TPB)
        def _hist():
            _zero(acc1_v)

            def body(i, _):
                idxs = [
                    idx_v[pl.ds((i * U + u) * LANES, LANES)]
                    for u in range(U)
                ]
                for u in range(U):
                    plsc.addupdate_scatter(acc1_v, [idxs[u]], ones16)
                return 0

            lax.fori_loop(0, NVEC // U, body, 0)
            pltpu.sync_copy(acc1_v, counts_hbm.at[b])

        out_cp = [None, None]
        for ci in range(CPG):
            p = ci % 2
            row = b * C + cg * CPG + ci
            grow = half * HB * C + row
            acc = accs[p]

            cp = pltpu.async_copy(
                feat_hbm.at[grow, pl.ds(0, CHUNK)], feats[0], fsem[0])
            if out_cp[p] is not None:
                out_cp[p].wait()
            if ci > 0:
                _zero(acc)

            for k in range(NCHUNK):
                if k + 1 < NCHUNK:
                    cp_next = pltpu.async_copy(
                        feat_hbm.at[grow, pl.ds((k + 1) * CHUNK, CHUNK)],
                        feats[(k + 1) % 2], fsem[(k + 1) % 2])
                cp.wait()
                fbuf = feats[k % 2]

                def body(i, _, k=k, fbuf=fbuf, acc=acc):
                    offs = [(i * U + u) * LANES for u in range(U)]
                    idxs = [
                        idx_v[pl.ds(k * CHUNK + off, LANES)] for off in offs
                    ]
                    vals = [fbuf[pl.ds(off, LANES)] for off in offs]
                    for u in range(U):
                        plsc.addupdate_scatter(acc, [idxs[u]], vals[u])
                    return 0

                lax.fori_loop(0, CHUNK // (LANES * U), body, 0)
                if k + 1 < NCHUNK:
                    cp = cp_next

            out_cp[p] = pltpu.async_copy(acc, sums_hbm.at[row], osem[p])

        for p in (0, 1):
            out_cp[p].wait()

    return sc_scatter


_sc_scatter_halves = [_make_sc_scatter(s) for s in range(SPLITS)]


VB = 32768


def _scale_transpose(sums_ref, counts_ref, out_ref):
    sb = sums_ref[...]
    cb = counts_ref[pl.ds(pl.program_id(0), 1), :]
    inv = 1.0 / jnp.maximum(cb, 1.0)
    scaled = sb * inv
    col = lax.broadcasted_iota(jnp.int32, (C, C), 0)
    rowi = lax.broadcasted_iota(jnp.int32, (C, C), 1)
    eye = (col == rowi).astype(jnp.float32)
    out_ref[0] = lax.dot_general(
        scaled, eye, (((0,), (0,)), ((), ())),
        preferred_element_type=jnp.float32)


def _finalize_first_body(sums_ref, counts_ref, out_ref):
    _scale_transpose(sums_ref, counts_ref, out_ref)


def _finalize_next_body(sums_ref, counts_ref, prev_ref, out_ref):
    del prev_ref
    _scale_transpose(sums_ref, counts_ref, out_ref)


def _stage_finalize(s, sums, counts, prev):
    off = s * HB
    if s == 0:
        return pl.pallas_call(
            _finalize_first_body,
            grid=(HB, NV // VB),
            in_specs=[
                pl.BlockSpec((C, VB), lambda b, i: (b, i)),
                pl.BlockSpec((HB, VB), lambda b, i: (0, i)),
            ],
            out_specs=pl.BlockSpec((1, VB, C), lambda b, i: (b, i, 0)),
            out_shape=jax.ShapeDtypeStruct((B, NV, C), jnp.float32),
        )(sums, counts)
    return pl.pallas_call(
        _finalize_next_body,
        grid=(HB, NV // VB),
        in_specs=[
            pl.BlockSpec((C, VB), lambda b, i: (b, i)),
            pl.BlockSpec((HB, VB), lambda b, i: (0, i)),
            pl.BlockSpec((1, 8, C), lambda b, i: (0, 0, 0)),
        ],
        out_specs=pl.BlockSpec((1, VB, C), lambda b, i, off=off: (b + off, i, 0)),
        out_shape=jax.ShapeDtypeStruct((B, NV, C), jnp.float32),
        input_output_aliases={2: 0},
    )(sums, counts, prev)


def kernel(features, coords):
    coords = lax.stop_gradient(coords)
    coords_t = jnp.transpose(coords, (1, 0, 2))
    nc_t, h = _stage_norm(coords_t)
    feat2 = features.reshape(B * C, N)
    parts = [f(feat2, h) for f in _sc_scatter_halves]
    out3 = None
    for s, (sums_s, counts_s) in enumerate(parts):
        out3 = _stage_finalize(s, sums_s, counts_s, out3)
    grid = jnp.transpose(out3.reshape(B, R, R, R, C), (0, 4, 1, 2, 3))
    nc = jnp.transpose(nc_t, (1, 0, 2))
    return grid, nc


# speedup vs baseline: 1.0048x; 1.0048x over previous
import functools

import jax
import jax.numpy as jnp
from jax import lax
from jax.experimental import pallas as pl
from jax.experimental.pallas import tpu as pltpu
from jax.experimental.pallas import tpu_sc as plsc

B, C, N, R = 8, 64, 32768, 32
NV = R * R * R
LANES = 16
NVEC = N // LANES
CHUNK = 8192
NCHUNK = N // CHUNK
SPLITS = 2
HB = B // SPLITS
NTEC = 32
TPB = NTEC // HB
CPG = C // TPB
U = 8


def _norm_body(coords_ref, nc_ref, h_ref):
    c = coords_ref[...]
    mean = jnp.mean(c, axis=2, keepdims=True)
    x = c - mean
    sumsq = jnp.sum(x * x, axis=0)
    denom = jnp.sqrt(jnp.max(sumsq, axis=1, keepdims=True)) * 2.0
    nc = x / denom[None] + 0.5
    nc = jnp.clip(nc * float(R), 0.0, float(R - 1))
    nc_ref[...] = nc
    vox = jnp.round(nc).astype(jnp.int32)
    h_ref[...] = vox[0] * (R * R) + vox[1] * R + vox[2]


def _stage_norm(coords_t):
    return pl.pallas_call(
        _norm_body,
        grid=(1,),
        in_specs=[pl.BlockSpec((3, B, N), lambda i: (0, 0, 0))],
        out_specs=[
            pl.BlockSpec((3, B, N), lambda i: (0, 0, 0)),
            pl.BlockSpec((B, N), lambda i: (0, 0)),
        ],
        out_shape=[
            jax.ShapeDtypeStruct((3, B, N), jnp.float32),
            jax.ShapeDtypeStruct((B, N), jnp.int32),
        ],
    )(coords_t)


_mesh = plsc.VectorSubcoreMesh(core_axis_name="c", subcore_axis_name="s")


def _make_sc_scatter(half):
    @functools.partial(
        pl.kernel,
        mesh=_mesh,
        out_type=[
            jax.ShapeDtypeStruct((HB * C, N), jnp.float32),
            jax.ShapeDtypeStruct((HB, NV), jnp.float32),
        ],
        compiler_params=pltpu.CompilerParams(needs_layout_passes=False),
        scratch_types=[
            pltpu.VMEM((N,), jnp.int32),
            pltpu.VMEM((CHUNK,), jnp.float32),
            pltpu.VMEM((CHUNK,), jnp.float32),
            pltpu.VMEM((NV,), jnp.float32),
            pltpu.VMEM((NV,), jnp.float32),
            pltpu.SemaphoreType.DMA,
            pltpu.SemaphoreType.DMA,
            pltpu.SemaphoreType.DMA,
            pltpu.SemaphoreType.DMA,
        ],
    )
    def sc_scatter(feat_hbm, h_hbm, sums_hbm, counts_hbm, idx_v, feat0_v,
                   feat1_v, acc0_v, acc1_v, fsem0, fsem1, osem0, osem1):
        wid = lax.axis_index("s") * 2 + lax.axis_index("c")
        b = wid // TPB
        cg = wid % TPB
        feats = [feat0_v, feat1_v]
        accs = [acc0_v, acc1_v]
        fsem = [fsem0, fsem1]
        osem = [osem0, osem1]

        pltpu.sync_copy(h_hbm.at[half * HB + b], idx_v)

        zeros16 = jnp.zeros((LANES,), jnp.float32)
        ones16 = jnp.ones((LANES,), jnp.float32)

        def _zero(ref):
            def body(i, _):
                for u in range(U):
                    ref[pl.ds((i * U + u) * LANES, LANES)] = zeros16
                return 0

            lax.fori_loop(0, NVEC // U, body, 0)

        _zero(acc0_v)

        @pl.when(cg == b % TPB)
        def _hist():
            _zero(acc1_v)

            def body(i, _):
                idxs = [
                    idx_v[pl.ds((i * U + u) * LANES, LANES)]
                    for u in range(U)
                ]
                for u in range(U):
                    plsc.addupdate_scatter(acc1_v, [idxs[u]], ones16)
                return 0

            lax.fori_loop(0, NVEC // U, body, 0)
            pltpu.sync_copy(acc1_v, counts_hbm.at[b])

        out_cp = [None, None]
        for ci in range(CPG):
            p = ci % 2
            row = b * C + cg * CPG + ci
            grow = half * HB * C + row
            acc = accs[p]

            cp = pltpu.async_copy(
                feat_hbm.at[grow, pl.ds(0, CHUNK)], feats[0], fsem[0])
            if out_cp[p] is not None:
                out_cp[p].wait()
            if ci > 0:
                _zero(acc)

            for k in range(NCHUNK):
                if k + 1 < NCHUNK:
                    cp_next = pltpu.async_copy(
                        feat_hbm.at[grow, pl.ds((k + 1) * CHUNK, CHUNK)],
                        feats[(k + 1) % 2], fsem[(k + 1) % 2])
                cp.wait()
                fbuf = feats[k % 2]

                def body(i, _, k=k, fbuf=fbuf, acc=acc):
                    offs = [(i * U + u) * LANES for u in range(U)]
                    idxs = [
                        idx_v[pl.ds(k * CHUNK + off, LANES)] for off in offs
                    ]
                    vals = [fbuf[pl.ds(off, LANES)] for off in offs]
                    for u in range(U):
                        plsc.addupdate_scatter(acc, [idxs[u]], vals[u])
                    return 0

                lax.fori_loop(0, CHUNK // (LANES * U), body, 0)
                if k + 1 < NCHUNK:
                    cp = cp_next

            out_cp[p] = pltpu.async_copy(acc, sums_hbm.at[row], osem[p])

        for p in (0, 1):
            out_cp[p].wait()

    return sc_scatter


_sc_scatter_halves = [_make_sc_scatter(s) for s in range(SPLITS)]


VB = 16384


def _scale_transpose(sums_ref, counts_ref, out_ref):
    sb = sums_ref[...]
    cb = counts_ref[pl.ds(pl.program_id(0), 1), :]
    inv = 1.0 / jnp.maximum(cb, 1.0)
    scaled = sb * inv
    col = lax.broadcasted_iota(jnp.int32, (C, C), 0)
    rowi = lax.broadcasted_iota(jnp.int32, (C, C), 1)
    eye = (col == rowi).astype(jnp.float32)
    out_ref[0] = lax.dot_general(
        scaled, eye, (((0,), (0,)), ((), ())),
        preferred_element_type=jnp.float32)


def _finalize_first_body(sums_ref, counts_ref, out_ref):
    _scale_transpose(sums_ref, counts_ref, out_ref)


def _finalize_next_body(sums_ref, counts_ref, prev_ref, out_ref):
    del prev_ref
    _scale_transpose(sums_ref, counts_ref, out_ref)


def _stage_finalize(s, sums, counts, prev):
    off = s * HB
    if s == 0:
        return pl.pallas_call(
            _finalize_first_body,
            grid=(HB, NV // VB),
            in_specs=[
                pl.BlockSpec((C, VB), lambda b, i: (b, i)),
                pl.BlockSpec((HB, VB), lambda b, i: (0, i)),
            ],
            out_specs=pl.BlockSpec((1, VB, C), lambda b, i: (b, i, 0)),
            out_shape=jax.ShapeDtypeStruct((B, NV, C), jnp.float32),
        )(sums, counts)
    return pl.pallas_call(
        _finalize_next_body,
        grid=(HB, NV // VB),
        in_specs=[
            pl.BlockSpec((C, VB), lambda b, i: (b, i)),
            pl.BlockSpec((HB, VB), lambda b, i: (0, i)),
            pl.BlockSpec((1, 8, C), lambda b, i: (0, 0, 0)),
        ],
        out_specs=pl.BlockSpec((1, VB, C), lambda b, i, off=off: (b + off, i, 0)),
        out_shape=jax.ShapeDtypeStruct((B, NV, C), jnp.float32),
        input_output_aliases={2: 0},
    )(sums, counts, prev)


def kernel(features, coords):
    coords = lax.stop_gradient(coords)
    coords_t = jnp.transpose(coords, (1, 0, 2))
    nc_t, h = _stage_norm(coords_t)
    feat2 = features.reshape(B * C, N)
    parts = [f(feat2, h) for f in _sc_scatter_halves]
    out3 = None
    for s, (sums_s, counts_s) in enumerate(parts):
        out3 = _stage_finalize(s, sums_s, counts_s, out3)
    grid = jnp.transpose(out3.reshape(B, R, R, R, C), (0, 4, 1, 2, 3))
    nc = jnp.transpose(nc_t, (1, 0, 2))
    return grid, nc
